# baseline (speedup 1.0000x reference)
import functools

import jax
import jax.numpy as jnp
from jax import lax
from jax.experimental import pallas as pl
from jax.experimental.pallas import tpu as pltpu
from jax.experimental.pallas import tpu_sc as plsc


def _sc_gather_mean(x_flat, table, B, S, D):
    info = plsc.get_sparse_core_info()
    NC, NS, L = info.num_cores, info.num_subcores, info.num_lanes
    NW = NC * NS
    assert B % NW == 0
    RPW = B // NW
    CB = 16
    assert RPW % CB == 0
    NCHUNK = RPW // CB
    NB = 2
    assert NCHUNK % NB == 0
    IDX = CB * S
    GSZ = 80
    assert IDX % GSZ == 0 and GSZ % 8 == 0
    NG = IDX // GSZ

    mesh = plsc.VectorSubcoreMesh(core_axis_name="c", subcore_axis_name="s")

    @functools.partial(
        pl.kernel,
        mesh=mesh,
        out_type=jax.ShapeDtypeStruct((B, D), jnp.float32),
        scratch_types=[
            pltpu.VMEM((RPW * S,), jnp.int32),
            pltpu.VMEM((IDX, D), jnp.float32),
            pltpu.VMEM((IDX, D), jnp.float32),
            pltpu.VMEM((CB, D), jnp.float32),
            pltpu.VMEM((CB, D), jnp.float32),
            pltpu.SemaphoreType.DMA,
            pltpu.SemaphoreType.DMA,
            pltpu.SemaphoreType.DMA,
            pltpu.SemaphoreType.DMA,
        ],
    )
    def k(x_hbm, tab_hbm, out_hbm, idx_all, rows0, rows1, pool0, pool1,
          sem0, sem1, osem0, osem1):
        bufs = ((rows0, sem0, pool0, osem0), (rows1, sem1, pool1, osem1))
        wid = lax.axis_index("s") * NC + lax.axis_index("c")
        base = wid * RPW
        inv = jnp.float32(1.0 / S)

        pltpu.sync_copy(x_hbm.at[pl.ds(base * S, RPW * S)], idx_all)

        def fire(c, rows_v, sem):
            for g in range(NG):
                pltpu.async_copy(
                    tab_hbm.at[idx_all.at[pl.ds(c * IDX + g * GSZ, GSZ)]],
                    rows_v.at[pl.ds(g * GSZ, GSZ)],
                    sem,
                )

        def drain(rows_v, sem):
            for g in range(NG):
                pltpu.make_async_copy(
                    tab_hbm.at[pl.ds(0, GSZ)],
                    rows_v.at[pl.ds(g * GSZ, GSZ)],
                    sem,
                ).wait()

        def reduce(c, rows_v, pool_v):
            @plsc.parallel_loop(0, CB, 1, unroll=4)
            def row_body(r):
                rb = r * S
                for j in range(D // L):
                    sl = pl.ds(j * L, L)
                    acc = rows_v[rb, sl]
                    for t in range(1, S):
                        acc = acc + rows_v[rb + t, sl]
                    pool_v[r, sl] = acc * inv

        def store(c, pool_v, osem):
            pltpu.async_copy(pool_v, out_hbm.at[pl.ds(base + c * CB, CB)], osem)

        def wait_store(c, pool_v, osem):
            pltpu.make_async_copy(
                pool_v, out_hbm.at[pl.ds(base + c * CB, CB)], osem
            ).wait()

        for b, (rv, sm, pv, osm) in enumerate(bufs):
            fire(b, rv, sm)

        for b, (rv, sm, pv, osm) in enumerate(bufs):
            drain(rv, sm)
            reduce(b, rv, pv)
            fire(b + NB, rv, sm)
            store(b, pv, osm)

        def outer(i, carry):
            for b, (rv, sm, pv, osm) in enumerate(bufs):
                c = i * NB + b
                drain(rv, sm)
                wait_store(c - NB, pv, osm)
                reduce(c, rv, pv)
                fire(c + NB, rv, sm)
                store(c, pv, osm)
            return carry

        lax.fori_loop(1, NCHUNK // NB - 1, outer, 0)

        last = NCHUNK - NB
        for b, (rv, sm, pv, osm) in enumerate(bufs):
            drain(rv, sm)
            wait_store(last + b - NB, pv, osm)
            reduce(last + b, rv, pv)
            store(last + b, pv, osm)
        for b, (rv, sm, pv, osm) in enumerate(bufs):
            wait_store(last + b, pv, osm)

    return k(x_flat, table)


def _tc_mlp(pooled, W1t, b1, W2t, b2):
    B, D = pooled.shape
    H = W1t.shape[1]
    C = W2t.shape[1]
    BM = 1024
    assert B % BM == 0

    def body(m_ref, w1_ref, b1_ref, w2_ref, b2_ref, o_ref):
        m = m_ref[...]
        h = jnp.dot(m, w1_ref[...], preferred_element_type=jnp.float32)
        h = jnp.maximum(h + b1_ref[...], 0.0)
        o = jnp.dot(h, w2_ref[...], preferred_element_type=jnp.float32)
        o = jnp.maximum(o + b2_ref[...], 0.0)
        mx = jnp.max(o, axis=1, keepdims=True)
        lse = jnp.log(jnp.sum(jnp.exp(o - mx), axis=1, keepdims=True)) + mx
        o_ref[...] = o - lse

    return pl.pallas_call(
        body,
        grid=(B // BM,),
        in_specs=[
            pl.BlockSpec((BM, D), lambda i: (i, 0)),
            pl.BlockSpec((D, H), lambda i: (0, 0)),
            pl.BlockSpec((1, H), lambda i: (0, 0)),
            pl.BlockSpec((H, C), lambda i: (0, 0)),
            pl.BlockSpec((1, C), lambda i: (0, 0)),
        ],
        out_specs=pl.BlockSpec((BM, C), lambda i: (i, 0)),
        out_shape=jax.ShapeDtypeStruct((B, C), jnp.float32),
    )(pooled, W1t, b1.reshape(1, H), W2t, b2.reshape(1, C))


def kernel(x, table, W1, b1, W2, b2):
    B, S = x.shape
    V, D = table.shape
    x_flat = x.reshape(-1).astype(jnp.int32)
    pooled = _sc_gather_mean(x_flat, table, B, S, D)
    return _tc_mlp(pooled, W1.T, b1, W2.T, b2)


# speedup vs baseline: 1.1588x; 1.1588x over previous
import functools

import jax
import jax.numpy as jnp
from jax import lax
from jax.experimental import pallas as pl
from jax.experimental.pallas import tpu as pltpu
from jax.experimental.pallas import tpu_sc as plsc


def _sc_gather_mean(x_flat, table, B, S, D):
    info = plsc.get_sparse_core_info()
    NC, NS, L = info.num_cores, info.num_subcores, info.num_lanes
    NW = NC * NS
    assert B % NW == 0
    RPW = B // NW
    CB = 16
    assert RPW % CB == 0
    NCHUNK = RPW // CB
    NB = 2
    assert NCHUNK % NB == 0
    IDX = CB * S
    GSZ = 40
    assert IDX % GSZ == 0 and GSZ % 8 == 0
    NG = IDX // GSZ

    mesh = plsc.VectorSubcoreMesh(core_axis_name="c", subcore_axis_name="s")

    @functools.partial(
        pl.kernel,
        mesh=mesh,
        out_type=jax.ShapeDtypeStruct((B, D), jnp.float32),
        scratch_types=[
            pltpu.VMEM((RPW * S,), jnp.int32),
            pltpu.VMEM((IDX, D), jnp.float32),
            pltpu.VMEM((IDX, D), jnp.float32),
            pltpu.VMEM((CB, D), jnp.float32),
            pltpu.VMEM((CB, D), jnp.float32),
            pltpu.SemaphoreType.DMA,
            pltpu.SemaphoreType.DMA,
            pltpu.SemaphoreType.DMA,
            pltpu.SemaphoreType.DMA,
        ],
    )
    def k(x_hbm, tab_hbm, out_hbm, idx_all, rows0, rows1, pool0, pool1,
          sem0, sem1, osem0, osem1):
        bufs = ((rows0, sem0, pool0, osem0), (rows1, sem1, pool1, osem1))
        wid = lax.axis_index("s") * NC + lax.axis_index("c")
        base = wid * RPW
        inv = jnp.float32(1.0 / S)

        pltpu.sync_copy(x_hbm.at[pl.ds(base * S, RPW * S)], idx_all)

        def fire(c, rows_v, sem):
            for g in range(NG):
                pltpu.async_copy(
                    tab_hbm.at[idx_all.at[pl.ds(c * IDX + g * GSZ, GSZ)]],
                    rows_v.at[pl.ds(g * GSZ, GSZ)],
                    sem,
                )

        def drain(rows_v, sem):
            for g in range(NG):
                pltpu.make_async_copy(
                    tab_hbm.at[pl.ds(0, GSZ)],
                    rows_v.at[pl.ds(g * GSZ, GSZ)],
                    sem,
                ).wait()

        def reduce(c, rows_v, pool_v):
            @plsc.parallel_loop(0, CB, 1, unroll=2)
            def row_body(r):
                rb = r * S
                for j in range(D // L):
                    sl = pl.ds(j * L, L)
                    acc = rows_v[rb, sl]
                    for t in range(1, S):
                        acc = acc + rows_v[rb + t, sl]
                    pool_v[r, sl] = acc * inv

        def store(c, pool_v, osem):
            pltpu.async_copy(pool_v, out_hbm.at[pl.ds(base + c * CB, CB)], osem)

        def wait_store(c, pool_v, osem):
            pltpu.make_async_copy(
                pool_v, out_hbm.at[pl.ds(base + c * CB, CB)], osem
            ).wait()

        for b, (rv, sm, pv, osm) in enumerate(bufs):
            fire(b, rv, sm)

        for b, (rv, sm, pv, osm) in enumerate(bufs):
            drain(rv, sm)
            reduce(b, rv, pv)
            fire(b + NB, rv, sm)
            store(b, pv, osm)

        def outer(i, carry):
            for b, (rv, sm, pv, osm) in enumerate(bufs):
                c = i * NB + b
                drain(rv, sm)
                wait_store(c - NB, pv, osm)
                reduce(c, rv, pv)
                fire(c + NB, rv, sm)
                store(c, pv, osm)
            return carry

        lax.fori_loop(1, NCHUNK // NB - 1, outer, 0)

        last = NCHUNK - NB
        for b, (rv, sm, pv, osm) in enumerate(bufs):
            drain(rv, sm)
            wait_store(last + b - NB, pv, osm)
            reduce(last + b, rv, pv)
            store(last + b, pv, osm)
        for b, (rv, sm, pv, osm) in enumerate(bufs):
            wait_store(last + b, pv, osm)

    return k(x_flat, table)


def _tc_mlp(pooled, W1t, b1, W2t, b2):
    B, D = pooled.shape
    H = W1t.shape[1]
    C = W2t.shape[1]
    BM = 1024
    assert B % BM == 0

    def body(m_ref, w1_ref, b1_ref, w2_ref, b2_ref, o_ref):
        m = m_ref[...]
        h = jnp.dot(m, w1_ref[...], preferred_element_type=jnp.float32)
        h = jnp.maximum(h + b1_ref[...], 0.0)
        o = jnp.dot(h, w2_ref[...], preferred_element_type=jnp.float32)
        o = jnp.maximum(o + b2_ref[...], 0.0)
        mx = jnp.max(o, axis=1, keepdims=True)
        lse = jnp.log(jnp.sum(jnp.exp(o - mx), axis=1, keepdims=True)) + mx
        o_ref[...] = o - lse

    return pl.pallas_call(
        body,
        grid=(B // BM,),
        in_specs=[
            pl.BlockSpec((BM, D), lambda i: (i, 0)),
            pl.BlockSpec((D, H), lambda i: (0, 0)),
            pl.BlockSpec((1, H), lambda i: (0, 0)),
            pl.BlockSpec((H, C), lambda i: (0, 0)),
            pl.BlockSpec((1, C), lambda i: (0, 0)),
        ],
        out_specs=pl.BlockSpec((BM, C), lambda i: (i, 0)),
        out_shape=jax.ShapeDtypeStruct((B, C), jnp.float32),
    )(pooled, W1t, b1.reshape(1, H), W2t, b2.reshape(1, C))


def kernel(x, table, W1, b1, W2, b2):
    B, S = x.shape
    V, D = table.shape
    x_flat = x.reshape(-1).astype(jnp.int32)
    pooled = _sc_gather_mean(x_flat, table, B, S, D)
    return _tc_mlp(pooled, W1.T, b1, W2.T, b2)


# speedup vs baseline: 1.1616x; 1.0025x over previous
import functools

import jax
import jax.numpy as jnp
from jax import lax
from jax.experimental import pallas as pl
from jax.experimental.pallas import tpu as pltpu
from jax.experimental.pallas import tpu_sc as plsc


def _sc_gather_mean(x_flat, table, B, S, D):
    info = plsc.get_sparse_core_info()
    NC, NS, L = info.num_cores, info.num_subcores, info.num_lanes
    NW = NC * NS
    assert B % NW == 0
    RPW = B // NW
    CB = 16
    assert RPW % CB == 0
    NCHUNK = RPW // CB
    NB = 2
    assert NCHUNK % NB == 0
    IDX = CB * S
    GSZ = 80
    assert IDX % GSZ == 0 and GSZ % 8 == 0
    NG = IDX // GSZ

    mesh = plsc.VectorSubcoreMesh(core_axis_name="c", subcore_axis_name="s")

    @functools.partial(
        pl.kernel,
        mesh=mesh,
        out_type=jax.ShapeDtypeStruct((B, D), jnp.float32),
        scratch_types=[
            pltpu.VMEM((RPW * S,), jnp.int32),
            pltpu.VMEM((IDX, D), jnp.float32),
            pltpu.VMEM((IDX, D), jnp.float32),
            pltpu.VMEM((CB, D), jnp.float32),
            pltpu.VMEM((CB, D), jnp.float32),
            pltpu.SemaphoreType.DMA,
            pltpu.SemaphoreType.DMA,
            pltpu.SemaphoreType.DMA,
            pltpu.SemaphoreType.DMA,
        ],
    )
    def k(x_hbm, tab_hbm, out_hbm, idx_all, rows0, rows1, pool0, pool1,
          sem0, sem1, osem0, osem1):
        bufs = ((rows0, sem0, pool0, osem0), (rows1, sem1, pool1, osem1))
        wid = lax.axis_index("s") * NC + lax.axis_index("c")
        base = wid * RPW
        inv = jnp.float32(1.0 / S)

        pltpu.sync_copy(x_hbm.at[pl.ds(base * S, RPW * S)], idx_all)

        def fire(c, rows_v, sem):
            for g in range(NG):
                pltpu.async_copy(
                    tab_hbm.at[idx_all.at[pl.ds(c * IDX + g * GSZ, GSZ)]],
                    rows_v.at[pl.ds(g * GSZ, GSZ)],
                    sem,
                )

        def drain(rows_v, sem):
            for g in range(NG):
                pltpu.make_async_copy(
                    tab_hbm.at[pl.ds(0, GSZ)],
                    rows_v.at[pl.ds(g * GSZ, GSZ)],
                    sem,
                ).wait()

        def reduce(c, rows_v, pool_v):
            @plsc.parallel_loop(0, CB, 1, unroll=2)
            def row_body(r):
                rb = r * S
                for j in range(D // L):
                    sl = pl.ds(j * L, L)
                    acc = rows_v[rb, sl]
                    for t in range(1, S):
                        acc = acc + rows_v[rb + t, sl]
                    pool_v[r, sl] = acc * inv

        def store(c, pool_v, osem):
            pltpu.async_copy(pool_v, out_hbm.at[pl.ds(base + c * CB, CB)], osem)

        def wait_store(c, pool_v, osem):
            pltpu.make_async_copy(
                pool_v, out_hbm.at[pl.ds(base + c * CB, CB)], osem
            ).wait()

        for b, (rv, sm, pv, osm) in enumerate(bufs):
            fire(b, rv, sm)

        for b, (rv, sm, pv, osm) in enumerate(bufs):
            drain(rv, sm)
            reduce(b, rv, pv)
            fire(b + NB, rv, sm)
            store(b, pv, osm)

        def outer(i, carry):
            for b, (rv, sm, pv, osm) in enumerate(bufs):
                c = i * NB + b
                drain(rv, sm)
                wait_store(c - NB, pv, osm)
                reduce(c, rv, pv)
                fire(c + NB, rv, sm)
                store(c, pv, osm)
            return carry

        lax.fori_loop(1, NCHUNK // NB - 1, outer, 0)

        last = NCHUNK - NB
        for b, (rv, sm, pv, osm) in enumerate(bufs):
            drain(rv, sm)
            wait_store(last + b - NB, pv, osm)
            reduce(last + b, rv, pv)
            store(last + b, pv, osm)
        for b, (rv, sm, pv, osm) in enumerate(bufs):
            wait_store(last + b, pv, osm)

    return k(x_flat, table)


def _tc_mlp(pooled, W1t, b1, W2t, b2):
    B, D = pooled.shape
    H = W1t.shape[1]
    C = W2t.shape[1]
    BM = 1024
    assert B % BM == 0

    def body(m_ref, w1_ref, b1_ref, w2_ref, b2_ref, o_ref):
        m = m_ref[...]
        h = jnp.dot(m, w1_ref[...], preferred_element_type=jnp.float32)
        h = jnp.maximum(h + b1_ref[...], 0.0)
        o = jnp.dot(h, w2_ref[...], preferred_element_type=jnp.float32)
        o = jnp.maximum(o + b2_ref[...], 0.0)
        mx = jnp.max(o, axis=1, keepdims=True)
        lse = jnp.log(jnp.sum(jnp.exp(o - mx), axis=1, keepdims=True)) + mx
        o_ref[...] = o - lse

    return pl.pallas_call(
        body,
        grid=(B // BM,),
        in_specs=[
            pl.BlockSpec((BM, D), lambda i: (i, 0)),
            pl.BlockSpec((D, H), lambda i: (0, 0)),
            pl.BlockSpec((1, H), lambda i: (0, 0)),
            pl.BlockSpec((H, C), lambda i: (0, 0)),
            pl.BlockSpec((1, C), lambda i: (0, 0)),
        ],
        out_specs=pl.BlockSpec((BM, C), lambda i: (i, 0)),
        out_shape=jax.ShapeDtypeStruct((B, C), jnp.float32),
    )(pooled, W1t, b1.reshape(1, H), W2t, b2.reshape(1, C))


def kernel(x, table, W1, b1, W2, b2):
    B, S = x.shape
    V, D = table.shape
    x_flat = x.reshape(-1).astype(jnp.int32)
    pooled = _sc_gather_mean(x_flat, table, B, S, D)
    return _tc_mlp(pooled, W1.T, b1, W2.T, b2)


# speedup vs baseline: 1.1853x; 1.0203x over previous
import functools

import jax
import jax.numpy as jnp
from jax import lax
from jax.experimental import pallas as pl
from jax.experimental.pallas import tpu as pltpu
from jax.experimental.pallas import tpu_sc as plsc


def _sc_gather_mean(x_flat, table, B, S, D):
    info = plsc.get_sparse_core_info()
    NC, NS, L = info.num_cores, info.num_subcores, info.num_lanes
    NW = NC * NS
    assert B % NW == 0
    RPW = B // NW
    CB = 16
    assert RPW % CB == 0
    NCHUNK = RPW // CB
    NB = 2
    assert NCHUNK % NB == 0
    IDX = CB * S
    GSZ = 80
    assert IDX % GSZ == 0 and GSZ % 8 == 0
    NG = IDX // GSZ

    mesh = plsc.VectorSubcoreMesh(core_axis_name="c", subcore_axis_name="s")

    @functools.partial(
        pl.kernel,
        mesh=mesh,
        out_type=jax.ShapeDtypeStruct((B, D), jnp.float32),
        scratch_types=[
            pltpu.VMEM((RPW * S,), jnp.int32),
            pltpu.VMEM((IDX, D), jnp.float32),
            pltpu.VMEM((IDX, D), jnp.float32),
            pltpu.VMEM((CB, D), jnp.float32),
            pltpu.VMEM((CB, D), jnp.float32),
            pltpu.SemaphoreType.DMA,
            pltpu.SemaphoreType.DMA,
            pltpu.SemaphoreType.DMA,
            pltpu.SemaphoreType.DMA,
        ],
    )
    def k(x_hbm, tab_hbm, out_hbm, idx_all, rows0, rows1, pool0, pool1,
          sem0, sem1, osem0, osem1):
        bufs = ((rows0, sem0, pool0, osem0), (rows1, sem1, pool1, osem1))
        wid = lax.axis_index("s") * NC + lax.axis_index("c")
        base = wid * RPW
        inv = jnp.float32(1.0 / S)

        pltpu.sync_copy(x_hbm.at[pl.ds(base * S, RPW * S)], idx_all)

        def fire(c, rows_v, sem):
            for g in range(NG):
                pltpu.async_copy(
                    tab_hbm.at[idx_all.at[pl.ds(c * IDX + g * GSZ, GSZ)]],
                    rows_v.at[pl.ds(g * GSZ, GSZ)],
                    sem,
                )

        def drain(rows_v, sem):
            for g in range(NG):
                pltpu.make_async_copy(
                    tab_hbm.at[pl.ds(0, GSZ)],
                    rows_v.at[pl.ds(g * GSZ, GSZ)],
                    sem,
                ).wait()

        def reduce(c, rows_v, pool_v):
            @plsc.parallel_loop(0, CB, 1, unroll=2)
            def row_body(r):
                rb = r * S
                for j in range(D // L):
                    sl = pl.ds(j * L, L)
                    acc = rows_v[rb, sl]
                    for t in range(1, S):
                        acc = acc + rows_v[rb + t, sl]
                    pool_v[r, sl] = acc * inv

        def store(c, pool_v, osem):
            pltpu.async_copy(pool_v, out_hbm.at[pl.ds(base + c * CB, CB)], osem)

        def wait_store(c, pool_v, osem):
            pltpu.make_async_copy(
                pool_v, out_hbm.at[pl.ds(base + c * CB, CB)], osem
            ).wait()

        for b, (rv, sm, pv, osm) in enumerate(bufs):
            fire(b, rv, sm)

        for b, (rv, sm, pv, osm) in enumerate(bufs):
            drain(rv, sm)
            reduce(b, rv, pv)
            fire(b + NB, rv, sm)
            store(b, pv, osm)

        def outer(i, carry):
            for b, (rv, sm, pv, osm) in enumerate(bufs):
                c = i * NB + b
                drain(rv, sm)
                wait_store(c - NB, pv, osm)
                reduce(c, rv, pv)
                fire(c + NB, rv, sm)
                store(c, pv, osm)
            return carry

        lax.fori_loop(1, NCHUNK // NB - 1, outer, 0)

        last = NCHUNK - NB
        for b, (rv, sm, pv, osm) in enumerate(bufs):
            drain(rv, sm)
            wait_store(last + b - NB, pv, osm)
            reduce(last + b, rv, pv)
            store(last + b, pv, osm)
        for b, (rv, sm, pv, osm) in enumerate(bufs):
            wait_store(last + b, pv, osm)

    return k(x_flat, table)


def _tc_mlp(pooled, W1t, b1, W2t, b2):
    B, D = pooled.shape
    H = W1t.shape[1]
    C = W2t.shape[1]
    BM = 2048
    assert B % BM == 0

    def body(m_ref, w1_ref, b1_ref, w2_ref, b2_ref, o_ref):
        m = m_ref[...]
        h = jnp.dot(m, w1_ref[...], preferred_element_type=jnp.float32)
        h = jnp.maximum(h + b1_ref[...], 0.0)
        o = jnp.dot(h, w2_ref[...], preferred_element_type=jnp.float32)
        o = jnp.maximum(o + b2_ref[...], 0.0)
        mx = jnp.max(o, axis=1, keepdims=True)
        lse = jnp.log(jnp.sum(jnp.exp(o - mx), axis=1, keepdims=True)) + mx
        o_ref[...] = o - lse

    return pl.pallas_call(
        body,
        grid=(B // BM,),
        in_specs=[
            pl.BlockSpec((BM, D), lambda i: (i, 0)),
            pl.BlockSpec((D, H), lambda i: (0, 0)),
            pl.BlockSpec((1, H), lambda i: (0, 0)),
            pl.BlockSpec((H, C), lambda i: (0, 0)),
            pl.BlockSpec((1, C), lambda i: (0, 0)),
        ],
        out_specs=pl.BlockSpec((BM, C), lambda i: (i, 0)),
        out_shape=jax.ShapeDtypeStruct((B, C), jnp.float32),
    )(pooled, W1t, b1.reshape(1, H), W2t, b2.reshape(1, C))


def kernel(x, table, W1, b1, W2, b2):
    B, S = x.shape
    V, D = table.shape
    x_flat = x.reshape(-1).astype(jnp.int32)
    pooled = _sc_gather_mean(x_flat, table, B, S, D)
    return _tc_mlp(pooled, W1.T, b1, W2.T, b2)


# speedup vs baseline: 1.1899x; 1.0039x over previous
import functools

import jax
import jax.numpy as jnp
from jax import lax
from jax.experimental import pallas as pl
from jax.experimental.pallas import tpu as pltpu
from jax.experimental.pallas import tpu_sc as plsc


def _sc_gather_mean(x_flat, table, B, S, D):
    info = plsc.get_sparse_core_info()
    NC, NS, L = info.num_cores, info.num_subcores, info.num_lanes
    NW = NC * NS
    assert B % NW == 0
    RPW = B // NW
    CB = 16
    assert RPW % CB == 0
    NCHUNK = RPW // CB
    NB = 2
    assert NCHUNK % NB == 0
    IDX = CB * S
    GSZ = 80
    assert IDX % GSZ == 0 and GSZ % 8 == 0
    NG = IDX // GSZ

    mesh = plsc.VectorSubcoreMesh(core_axis_name="c", subcore_axis_name="s")

    @functools.partial(
        pl.kernel,
        mesh=mesh,
        out_type=jax.ShapeDtypeStruct((B, D), jnp.float32),
        scratch_types=[
            pltpu.VMEM((RPW * S,), jnp.int32),
            pltpu.VMEM((IDX, D), jnp.float32),
            pltpu.VMEM((IDX, D), jnp.float32),
            pltpu.VMEM((CB, D), jnp.float32),
            pltpu.VMEM((CB, D), jnp.float32),
            pltpu.SemaphoreType.DMA,
            pltpu.SemaphoreType.DMA,
            pltpu.SemaphoreType.DMA,
            pltpu.SemaphoreType.DMA,
        ],
    )
    def k(x_hbm, tab_hbm, out_hbm, idx_all, rows0, rows1, pool0, pool1,
          sem0, sem1, osem0, osem1):
        bufs = ((rows0, sem0, pool0, osem0), (rows1, sem1, pool1, osem1))
        wid = lax.axis_index("s") * NC + lax.axis_index("c")
        base = wid * RPW
        inv = jnp.float32(1.0 / S)

        pltpu.sync_copy(x_hbm.at[pl.ds(base * S, RPW * S)], idx_all)

        def fire(c, rows_v, sem):
            for g in range(NG):
                pltpu.async_copy(
                    tab_hbm.at[idx_all.at[pl.ds(c * IDX + g * GSZ, GSZ)]],
                    rows_v.at[pl.ds(g * GSZ, GSZ)],
                    sem,
                )

        def drain(rows_v, sem):
            for g in range(NG):
                pltpu.make_async_copy(
                    tab_hbm.at[pl.ds(0, GSZ)],
                    rows_v.at[pl.ds(g * GSZ, GSZ)],
                    sem,
                ).wait()

        def reduce(c, rows_v, pool_v):
            @plsc.parallel_loop(0, CB, 1, unroll=2)
            def row_body(r):
                rb = r * S
                for j in range(D // L):
                    sl = pl.ds(j * L, L)
                    acc = rows_v[rb, sl]
                    for t in range(1, S):
                        acc = acc + rows_v[rb + t, sl]
                    pool_v[r, sl] = acc * inv

        def store(c, pool_v, osem):
            pltpu.async_copy(pool_v, out_hbm.at[pl.ds(base + c * CB, CB)], osem)

        def wait_store(c, pool_v, osem):
            pltpu.make_async_copy(
                pool_v, out_hbm.at[pl.ds(base + c * CB, CB)], osem
            ).wait()

        for b, (rv, sm, pv, osm) in enumerate(bufs):
            fire(b, rv, sm)

        for b, (rv, sm, pv, osm) in enumerate(bufs):
            drain(rv, sm)
            reduce(b, rv, pv)
            fire(b + NB, rv, sm)
            store(b, pv, osm)

        def outer(i, carry):
            for b, (rv, sm, pv, osm) in enumerate(bufs):
                c = i * NB + b
                drain(rv, sm)
                wait_store(c - NB, pv, osm)
                reduce(c, rv, pv)
                fire(c + NB, rv, sm)
                store(c, pv, osm)
            return carry

        lax.fori_loop(1, NCHUNK // NB - 1, outer, 0)

        last = NCHUNK - NB
        for b, (rv, sm, pv, osm) in enumerate(bufs):
            drain(rv, sm)
            wait_store(last + b - NB, pv, osm)
            reduce(last + b, rv, pv)
            store(last + b, pv, osm)
        for b, (rv, sm, pv, osm) in enumerate(bufs):
            wait_store(last + b, pv, osm)

    return k(x_flat, table)


def _tc_mlp(pooled, W1t, b1, W2t, b2):
    B, D = pooled.shape
    H = W1t.shape[1]
    C = W2t.shape[1]
    BM = 4096
    assert B % BM == 0

    def body(m_ref, w1_ref, b1_ref, w2_ref, b2_ref, o_ref):
        m = m_ref[...]
        h = jnp.dot(m, w1_ref[...], preferred_element_type=jnp.float32)
        h = jnp.maximum(h + b1_ref[...], 0.0)
        o = jnp.dot(h, w2_ref[...], preferred_element_type=jnp.float32)
        o = jnp.maximum(o + b2_ref[...], 0.0)
        mx = jnp.max(o, axis=1, keepdims=True)
        lse = jnp.log(jnp.sum(jnp.exp(o - mx), axis=1, keepdims=True)) + mx
        o_ref[...] = o - lse

    return pl.pallas_call(
        body,
        grid=(B // BM,),
        in_specs=[
            pl.BlockSpec((BM, D), lambda i: (i, 0)),
            pl.BlockSpec((D, H), lambda i: (0, 0)),
            pl.BlockSpec((1, H), lambda i: (0, 0)),
            pl.BlockSpec((H, C), lambda i: (0, 0)),
            pl.BlockSpec((1, C), lambda i: (0, 0)),
        ],
        out_specs=pl.BlockSpec((BM, C), lambda i: (i, 0)),
        out_shape=jax.ShapeDtypeStruct((B, C), jnp.float32),
    )(pooled, W1t, b1.reshape(1, H), W2t, b2.reshape(1, C))


def kernel(x, table, W1, b1, W2, b2):
    B, S = x.shape
    V, D = table.shape
    x_flat = x.reshape(-1).astype(jnp.int32)
    pooled = _sc_gather_mean(x_flat, table, B, S, D)
    return _tc_mlp(pooled, W1.T, b1, W2.T, b2)
